# baseline (speedup 1.0000x reference)
import jax
import jax.numpy as jnp
from jax import lax
from jax.experimental import pallas as pl
from jax.experimental.pallas import tpu as pltpu
from jax.experimental.pallas import tpu_sc as plsc

N_NODES = 10000
N_EDGES = 320000
DELTA = 4.0
ROW_BLK = 400

NW = 32
NPW = 320
RING = 64
EW = 128
E_TOT = N_EDGES + N_NODES
E_PAD = ((E_TOT + EW - 1) // EW) * EW
PTR_PAD = (NW - 1) * NPW + 352

_NEG = -3.0e38
_POS = 3.0e38


def _sc_body(h_hbm, src_hbm, wlast_hbm, ptr_hbm,
             amax_hbm, amin_hbm, s1_hbm, s2_hbm, cnt_hbm,
             idx0, idx1, idx2, idx3, rows0, rows1, rows2, rows3,
             ring0, ring1, ring2, ring3, cnt_v,
             ptr_v, wl0, wl1, wl2, wl3,
             sem0, sem1, sem2, sem3, wsem0, wsem1, wsem2, wsem3):
    w = lax.axis_index("s") * 2 + lax.axis_index("c")
    base = w * NPW
    pltpu.sync_copy(ptr_hbm.at[pl.ds(pl.multiple_of(base, 8), 352)], ptr_v)

    def ptr_at(k):
        return ptr_v[pl.ds(k, 16)][0]

    lo = ptr_at(0)
    nn = jnp.where(w == NW - 1, N_NODES - (NW - 1) * NPW, NPW)
    hi = ptr_at(nn)
    g0 = lo // EW
    g1 = (hi + EW - 1) // EW
    nwin = g1 - g0

    rings = (ring0, ring1, ring2, ring3)
    outs = (amax_hbm, amin_hbm, s1_hbm, s2_hbm)
    bufs = ((idx0, rows0, sem0, wl0, wsem0),
            (idx1, rows1, sem1, wl1, wsem1),
            (idx2, rows2, sem2, wl2, wsem2),
            (idx3, rows3, sem3, wl3, wsem3))

    ident = (tuple(jnp.full((16,), _NEG, jnp.float32) for _ in range(8))
             + tuple(jnp.full((16,), _POS, jnp.float32) for _ in range(8))
             + tuple(jnp.zeros((16,), jnp.float32) for _ in range(16)))

    def issue(g, idxb, rowsb, semb, wlb, wsemb):
        e0 = pl.multiple_of(g * EW, EW)
        pltpu.sync_copy(src_hbm.at[pl.ds(e0, EW)], idxb)
        pltpu.async_copy(h_hbm.at[idxb], rowsb, semb)
        woff = pl.multiple_of((g // 8) * 8, 8)
        pltpu.async_copy(wlast_hbm.at[pl.ds(woff, 16)], wlb, wsemb)

    def store_segment(ld, accs):
        slot = lax.rem(ld, RING)
        for a in range(4):
            for cg in range(8):
                rings[a][pl.ds(slot * 128 + cg * 16, 16)] = accs[a * 8 + cg]
        ccnt = (ptr_at(ld + 1) - ptr_at(ld)).astype(jnp.float32)
        cnt_v[pl.ds(ld * 16, 16)] = jnp.full((16,), 1.0, jnp.float32) * ccnt

        @pl.when(slot == RING - 1)
        def _():
            n0 = pl.multiple_of((base + ld - (RING - 1)) * 128, RING * 128)
            for a in range(4):
                pltpu.sync_copy(rings[a], outs[a].at[pl.ds(n0, RING * 128)])

    for _k in range(4):
        @pl.when(nwin > _k)
        def _(_k=_k):
            issue(g0 + _k, *bufs[_k])

    def process(g, idxb, rowsb, semb, wlb, wsemb, carry):
        @pl.when(g < g1)
        def _():
            pltpu.make_async_copy(h_hbm.at[idxb], rowsb, semb).wait()
            pltpu.make_async_copy(wlast_hbm.at[pl.ds(0, 16)], wlb,
                                  wsemb).wait()

        ebase = g * EW
        e_hi = jnp.minimum(hi, ebase + EW)
        estart = jnp.maximum(lo, ebase)

        nend = jnp.minimum(wlb[pl.ds(g - (g // 8) * 8, 16)][0],
                           base + nn - 1)

        n0 = carry[0]
        trips = nend - n0 + 1

        def node_body(i, c):
            accs = list(c[1:])
            ni = n0 + i
            ldi = ni - base
            p0 = ptr_at(ldi)
            pn1 = ptr_at(ldi + 1)
            s_i = jnp.maximum(p0, estart)
            t_i = jnp.minimum(pn1, e_hi)

            def acc_edge(e_, accs_t):
                accs_ = list(accs_t)
                ew_ = e_ - ebase
                for cg in range(8):
                    m = rowsb[ew_, pl.ds(cg * 16, 16)]
                    accs_[cg] = jnp.maximum(accs_[cg], m)
                    accs_[8 + cg] = jnp.minimum(accs_[8 + cg], m)
                    accs_[16 + cg] = accs_[16 + cg] + m
                    accs_[24 + cg] = accs_[24 + cg] + m * m
                return tuple(accs_)

            accs = lax.fori_loop(s_i, t_i, acc_edge, tuple(accs))

            done = t_i == pn1

            @pl.when(done)
            def _():
                store_segment(ldi, accs)

            n2 = jnp.where(done, ni + 1, ni)
            accs2 = tuple(
                jnp.where(done, iv, av) for iv, av in zip(ident, accs))
            return (n2,) + accs2

        carry = lax.fori_loop(0, trips, node_body, carry)

        @pl.when(g + 4 < g1)
        def _():
            issue(g + 4, idxb, rowsb, semb, wlb, wsemb)

        return carry

    def quad_body(gp, carry):
        for k in range(4):
            carry = process(g0 + 4 * gp + k, *bufs[k], carry)
        return carry

    carry0 = (base,) + ident
    lax.fori_loop(0, (nwin + 3) // 4, quad_body, carry0)

    @pl.when(w == NW - 1)
    def _():
        for a in range(4):
            pltpu.sync_copy(rings[a].at[pl.ds(0, 16 * 128)],
                            outs[a].at[pl.ds((N_NODES - 16) * 128, 16 * 128)])
        pltpu.sync_copy(cnt_v.at[pl.ds(0, 80 * 16)],
                        cnt_hbm.at[pl.ds((NW - 1) * NPW * 16, 80 * 16)])

    @pl.when(w < NW - 1)
    def _():
        pltpu.sync_copy(
            cnt_v, cnt_hbm.at[pl.ds(pl.multiple_of(base * 16, NPW * 16),
                                    NPW * 16)])


@jax.jit
def _sc_aggregate(h, src_p, wlast, rowptr):
    f32 = jnp.float32
    out_type = (jax.ShapeDtypeStruct((N_NODES * 128,), f32),
                jax.ShapeDtypeStruct((N_NODES * 128,), f32),
                jax.ShapeDtypeStruct((N_NODES * 128,), f32),
                jax.ShapeDtypeStruct((N_NODES * 128,), f32),
                jax.ShapeDtypeStruct((N_NODES * 16,), f32))
    scratch = (
        [pltpu.VMEM((EW,), jnp.int32) for _ in range(4)]
        + [pltpu.VMEM((EW, 128), f32) for _ in range(4)]
        + [pltpu.VMEM((RING * 128,), f32) for _ in range(4)]
        + [pltpu.VMEM((NPW * 16,), f32),
           pltpu.VMEM((352,), jnp.int32)]
        + [pltpu.VMEM((16,), jnp.int32) for _ in range(4)]
        + [pltpu.SemaphoreType.DMA for _ in range(8)]
    )
    mesh = plsc.VectorSubcoreMesh(core_axis_name="c", subcore_axis_name="s")
    return pl.kernel(_sc_body, out_type=out_type, mesh=mesh,
                     scratch_types=scratch)(h, src_p, wlast, rowptr)


def _dense_body(amax_ref, amin_ref, s1_ref, s2_ref, cnt_ref, w_ref, b_ref,
                g_ref, be_ref, out_ref):
    amax = amax_ref[...]
    amin = amin_ref[...]
    s1 = s1_ref[...]
    s2 = s2_ref[...]
    cnt = cnt_ref[...]
    mean = s1 / cnt
    var = jnp.maximum(s2 / cnt - mean * mean, 0.0)
    std = jnp.sqrt(var + 1e-5)
    aggs = jnp.concatenate([amax, amin, std, var], axis=1)
    logd = jnp.log(cnt + 1.0)
    t1 = logd / DELTA
    t2 = DELTA / logd
    scaled = jnp.concatenate([aggs, aggs * t1[:, :1], aggs * t2[:, :1]], axis=1)
    h = lax.dot_general(scaled, w_ref[...], (((1,), (0,)), ((), ())),
                        preferred_element_type=jnp.float32)
    h = h + b_ref[...]
    mu = jnp.mean(h, axis=-1, keepdims=True)
    v = jnp.var(h, axis=-1, keepdims=True)
    h = (h - mu) / jnp.sqrt(v + 1e-5) * g_ref[...] + be_ref[...]
    out_ref[...] = jnp.where(h > 0, h, jnp.exp(h) - 1.0)


@jax.jit
def _dense_layer(amax, amin, s1, s2, cnt2d, W, b, g, be):
    grid = (N_NODES // ROW_BLK,)
    node_spec = pl.BlockSpec((ROW_BLK, 128), lambda i: (i, 0))
    return pl.pallas_call(
        _dense_body,
        grid=grid,
        in_specs=[
            node_spec, node_spec, node_spec, node_spec, node_spec,
            pl.BlockSpec((W.shape[0], 128), lambda i: (0, 0)),
            pl.BlockSpec((1, 128), lambda i: (0, 0)),
            pl.BlockSpec((1, 128), lambda i: (0, 0)),
            pl.BlockSpec((1, 128), lambda i: (0, 0)),
        ],
        out_specs=node_spec,
        out_shape=jax.ShapeDtypeStruct((N_NODES, 128), jnp.float32),
    )(amax, amin, s1, s2, cnt2d, W, b.reshape(1, 128), g.reshape(1, 128),
      be.reshape(1, 128))


def kernel(x, edge_index, W0, b0, g0, be0, W1, b1, g1, be1, W2, b2, g2, be2):
    loops = jnp.arange(N_NODES, dtype=edge_index.dtype)
    src = jnp.concatenate([edge_index[0], loops])
    dst = jnp.concatenate([edge_index[1], loops])
    key = lax.sort(dst * 16384 + src)
    dst_s = key >> 14
    src_s = key & 16383
    src_p = jnp.concatenate(
        [src_s, jnp.zeros((E_PAD - E_TOT,), src_s.dtype)])
    wl_idx = jnp.minimum(
        (jnp.arange(E_PAD // EW, dtype=jnp.int32) + 1) * EW, E_TOT) - 1
    wlast = dst_s[wl_idx].astype(jnp.int32)
    wlast = jnp.concatenate([wlast, jnp.zeros((16,), jnp.int32)])
    rowptr = jnp.searchsorted(dst_s, jnp.arange(N_NODES + 1, dtype=dst.dtype),
                              side="left", method="sort").astype(jnp.int32)
    rowptr = jnp.concatenate(
        [rowptr, jnp.full((PTR_PAD - (N_NODES + 1),), E_TOT, jnp.int32)])

    h = x
    cnt2d = None
    for (W, b, g, be) in ((W0, b0, g0, be0), (W1, b1, g1, be1),
                          (W2, b2, g2, be2)):
        amax, amin, s1, s2, cnt = _sc_aggregate(h, src_p, wlast, rowptr)
        amax = amax.reshape(N_NODES, 128)
        amin = amin.reshape(N_NODES, 128)
        s1 = s1.reshape(N_NODES, 128)
        s2 = s2.reshape(N_NODES, 128)
        if cnt2d is None:
            cnt2d = jnp.broadcast_to(cnt.reshape(N_NODES, 16)[:, :1],
                                     (N_NODES, 128))
        h = _dense_layer(amax, amin, s1, s2, cnt2d, W, b, g, be)
    return h


# speedup vs baseline: 2.6818x; 2.6818x over previous
import jax
import jax.numpy as jnp
from jax import lax
from jax.experimental import pallas as pl
from jax.experimental.pallas import tpu as pltpu
from jax.experimental.pallas import tpu_sc as plsc

N_NODES = 10000
N_EDGES = 320000
DELTA = 4.0
ROW_BLK = 400

NW = 32
NPW = 320
RING = 64
EW = 128
E_TOT = N_EDGES + N_NODES
E_PAD = ((E_TOT + EW - 1) // EW) * EW
PTR_PAD = (NW - 1) * NPW + 352

_NEG = -3.0e38
_POS = 3.0e38

CH = 10320
KEY_PAD = 31 * CH + 10384
RP_OUT = 10240


def _rp_body(key_hbm, out_hbm, key_v, zer_v, idx_st, val_st, shared, sem):
    c = lax.axis_index("c")
    sid = lax.axis_index("s")
    w = sid * 2 + c
    c0 = w * CH

    for i in range(40):
        zer_v[pl.ds(i * 16, 16)] = jnp.zeros((16,), jnp.int32)
    pltpu.sync_copy(zer_v, shared.at[pl.ds(sid * 640, 640)])
    plsc.subcore_barrier()

    pltpu.sync_copy(key_hbm.at[pl.ds(pl.multiple_of(c0, 16), 10384)],
                    key_v.at[pl.ds(0, 10384)])

    lane = lax.broadcasted_iota(jnp.int32, (16,), 0)
    for b in range(81):
        for v in range(8):
            i = b * 8 + v
            d0 = key_v[pl.ds(i * 16, 16)] >> 14
            d1 = key_v[pl.ds(i * 16 + 1, 16)] >> 14
            epos = i * 16 + lane
            mask = (d1 != d0) & (epos < CH)
            idx_st[pl.ds(v * 16, 16)] = jnp.where(mask, d1, 10008)
            val_st[pl.ds(v * 16, 16)] = jnp.where(mask, c0 + epos + 1, 0)
        pltpu.sync_copy(val_st, shared.at[idx_st], add=True)
    plsc.subcore_barrier()

    @pl.when(sid == 0)
    def _():
        pltpu.sync_copy(shared.at[pl.ds(0, RP_OUT)],
                        out_hbm.at[pl.ds(c * RP_OUT, RP_OUT)])


@jax.jit
def _rowptr_sc(key_p):
    scratch = [
        pltpu.VMEM((10400,), jnp.int32),
        pltpu.VMEM((640,), jnp.int32),
        pltpu.VMEM((128,), jnp.int32),
        pltpu.VMEM((128,), jnp.int32),
        pltpu.VMEM_SHARED((RP_OUT,), jnp.int32),
        pltpu.SemaphoreType.DMA,
    ]
    mesh = plsc.VectorSubcoreMesh(core_axis_name="c", subcore_axis_name="s")
    out_type = jax.ShapeDtypeStruct((2 * RP_OUT,), jnp.int32)
    return pl.kernel(_rp_body, out_type=out_type, mesh=mesh,
                     scratch_types=scratch)(key_p)


def _sc_body(h_hbm, src_hbm, wlast_hbm, ptr_hbm,
             amax_hbm, amin_hbm, s1_hbm, s2_hbm, cnt_hbm,
             idx0, idx1, idx2, idx3, rows0, rows1, rows2, rows3,
             ring0, ring1, ring2, ring3, cnt_v,
             ptr_v, wl0, wl1, wl2, wl3,
             sem0, sem1, sem2, sem3, wsem0, wsem1, wsem2, wsem3):
    w = lax.axis_index("s") * 2 + lax.axis_index("c")
    base = w * NPW
    pltpu.sync_copy(ptr_hbm.at[pl.ds(pl.multiple_of(base, 8), 352)], ptr_v)

    def ptr_at(k):
        return ptr_v[pl.ds(k, 16)][0]

    lo = ptr_at(0)
    nn = jnp.where(w == NW - 1, N_NODES - (NW - 1) * NPW, NPW)
    hi = ptr_at(nn)
    g0 = lo // EW
    g1 = (hi + EW - 1) // EW
    nwin = g1 - g0

    rings = (ring0, ring1, ring2, ring3)
    outs = (amax_hbm, amin_hbm, s1_hbm, s2_hbm)
    bufs = ((idx0, rows0, sem0, wl0, wsem0),
            (idx1, rows1, sem1, wl1, wsem1),
            (idx2, rows2, sem2, wl2, wsem2),
            (idx3, rows3, sem3, wl3, wsem3))

    ident = (tuple(jnp.full((16,), _NEG, jnp.float32) for _ in range(8))
             + tuple(jnp.full((16,), _POS, jnp.float32) for _ in range(8))
             + tuple(jnp.zeros((16,), jnp.float32) for _ in range(16)))

    def issue(g, idxb, rowsb, semb, wlb, wsemb):
        e0 = pl.multiple_of(g * EW, EW)
        pltpu.sync_copy(src_hbm.at[pl.ds(e0, EW)], idxb)
        pltpu.async_copy(h_hbm.at[idxb], rowsb, semb)
        woff = pl.multiple_of((g // 8) * 8, 8)
        pltpu.async_copy(wlast_hbm.at[pl.ds(woff, 16)], wlb, wsemb)

    def store_segment(ld, accs):
        slot = lax.rem(ld, RING)
        for a in range(4):
            for cg in range(8):
                rings[a][pl.ds(slot * 128 + cg * 16, 16)] = accs[a * 8 + cg]
        ccnt = (ptr_at(ld + 1) - ptr_at(ld)).astype(jnp.float32)
        cnt_v[pl.ds(ld * 16, 16)] = jnp.full((16,), 1.0, jnp.float32) * ccnt

        @pl.when(slot == RING - 1)
        def _():
            n0 = pl.multiple_of((base + ld - (RING - 1)) * 128, RING * 128)
            for a in range(4):
                pltpu.sync_copy(rings[a], outs[a].at[pl.ds(n0, RING * 128)])

    for _k in range(4):
        @pl.when(nwin > _k)
        def _(_k=_k):
            issue(g0 + _k, *bufs[_k])

    def process(g, idxb, rowsb, semb, wlb, wsemb, carry):
        @pl.when(g < g1)
        def _():
            pltpu.make_async_copy(h_hbm.at[idxb], rowsb, semb).wait()
            pltpu.make_async_copy(wlast_hbm.at[pl.ds(0, 16)], wlb,
                                  wsemb).wait()

        ebase = g * EW
        e_hi = jnp.minimum(hi, ebase + EW)
        estart = jnp.maximum(lo, ebase)

        nend = jnp.minimum(wlb[pl.ds(g - (g // 8) * 8, 16)][0],
                           base + nn - 1)

        n0 = carry[0]
        trips = nend - n0 + 1

        def node_body(i, c):
            accs = list(c[1:])
            ni = n0 + i
            ldi = ni - base
            p0 = ptr_at(ldi)
            pn1 = ptr_at(ldi + 1)
            s_i = jnp.maximum(p0, estart)
            t_i = jnp.minimum(pn1, e_hi)

            def acc_edge(e_, accs_t):
                accs_ = list(accs_t)
                ew_ = e_ - ebase
                for cg in range(8):
                    m = rowsb[ew_, pl.ds(cg * 16, 16)]
                    accs_[cg] = jnp.maximum(accs_[cg], m)
                    accs_[8 + cg] = jnp.minimum(accs_[8 + cg], m)
                    accs_[16 + cg] = accs_[16 + cg] + m
                    accs_[24 + cg] = accs_[24 + cg] + m * m
                return tuple(accs_)

            accs = lax.fori_loop(s_i, t_i, acc_edge, tuple(accs))

            done = t_i == pn1

            @pl.when(done)
            def _():
                store_segment(ldi, accs)

            n2 = jnp.where(done, ni + 1, ni)
            accs2 = tuple(
                jnp.where(done, iv, av) for iv, av in zip(ident, accs))
            return (n2,) + accs2

        carry = lax.fori_loop(0, trips, node_body, carry)

        @pl.when(g + 4 < g1)
        def _():
            issue(g + 4, idxb, rowsb, semb, wlb, wsemb)

        return carry

    def quad_body(gp, carry):
        for k in range(4):
            carry = process(g0 + 4 * gp + k, *bufs[k], carry)
        return carry

    carry0 = (base,) + ident
    lax.fori_loop(0, (nwin + 3) // 4, quad_body, carry0)

    @pl.when(w == NW - 1)
    def _():
        for a in range(4):
            pltpu.sync_copy(rings[a].at[pl.ds(0, 16 * 128)],
                            outs[a].at[pl.ds((N_NODES - 16) * 128, 16 * 128)])
        pltpu.sync_copy(cnt_v.at[pl.ds(0, 80 * 16)],
                        cnt_hbm.at[pl.ds((NW - 1) * NPW * 16, 80 * 16)])

    @pl.when(w < NW - 1)
    def _():
        pltpu.sync_copy(
            cnt_v, cnt_hbm.at[pl.ds(pl.multiple_of(base * 16, NPW * 16),
                                    NPW * 16)])


@jax.jit
def _sc_aggregate(h, src_p, wlast, rowptr):
    f32 = jnp.float32
    out_type = (jax.ShapeDtypeStruct((N_NODES * 128,), f32),
                jax.ShapeDtypeStruct((N_NODES * 128,), f32),
                jax.ShapeDtypeStruct((N_NODES * 128,), f32),
                jax.ShapeDtypeStruct((N_NODES * 128,), f32),
                jax.ShapeDtypeStruct((N_NODES * 16,), f32))
    scratch = (
        [pltpu.VMEM((EW,), jnp.int32) for _ in range(4)]
        + [pltpu.VMEM((EW, 128), f32) for _ in range(4)]
        + [pltpu.VMEM((RING * 128,), f32) for _ in range(4)]
        + [pltpu.VMEM((NPW * 16,), f32),
           pltpu.VMEM((352,), jnp.int32)]
        + [pltpu.VMEM((16,), jnp.int32) for _ in range(4)]
        + [pltpu.SemaphoreType.DMA for _ in range(8)]
    )
    mesh = plsc.VectorSubcoreMesh(core_axis_name="c", subcore_axis_name="s")
    return pl.kernel(_sc_body, out_type=out_type, mesh=mesh,
                     scratch_types=scratch)(h, src_p, wlast, rowptr)


def _dense_body(amax_ref, amin_ref, s1_ref, s2_ref, cnt_ref, w_ref, b_ref,
                g_ref, be_ref, out_ref):
    amax = amax_ref[...]
    amin = amin_ref[...]
    s1 = s1_ref[...]
    s2 = s2_ref[...]
    cnt = cnt_ref[...]
    mean = s1 / cnt
    var = jnp.maximum(s2 / cnt - mean * mean, 0.0)
    std = jnp.sqrt(var + 1e-5)
    aggs = jnp.concatenate([amax, amin, std, var], axis=1)
    logd = jnp.log(cnt + 1.0)
    t1 = logd / DELTA
    t2 = DELTA / logd
    scaled = jnp.concatenate([aggs, aggs * t1[:, :1], aggs * t2[:, :1]], axis=1)
    h = lax.dot_general(scaled, w_ref[...], (((1,), (0,)), ((), ())),
                        preferred_element_type=jnp.float32)
    h = h + b_ref[...]
    mu = jnp.mean(h, axis=-1, keepdims=True)
    v = jnp.var(h, axis=-1, keepdims=True)
    h = (h - mu) / jnp.sqrt(v + 1e-5) * g_ref[...] + be_ref[...]
    out_ref[...] = jnp.where(h > 0, h, jnp.exp(h) - 1.0)


@jax.jit
def _dense_layer(amax, amin, s1, s2, cnt2d, W, b, g, be):
    grid = (N_NODES // ROW_BLK,)
    node_spec = pl.BlockSpec((ROW_BLK, 128), lambda i: (i, 0))
    return pl.pallas_call(
        _dense_body,
        grid=grid,
        in_specs=[
            node_spec, node_spec, node_spec, node_spec, node_spec,
            pl.BlockSpec((W.shape[0], 128), lambda i: (0, 0)),
            pl.BlockSpec((1, 128), lambda i: (0, 0)),
            pl.BlockSpec((1, 128), lambda i: (0, 0)),
            pl.BlockSpec((1, 128), lambda i: (0, 0)),
        ],
        out_specs=node_spec,
        out_shape=jax.ShapeDtypeStruct((N_NODES, 128), jnp.float32),
    )(amax, amin, s1, s2, cnt2d, W, b.reshape(1, 128), g.reshape(1, 128),
      be.reshape(1, 128))


def kernel(x, edge_index, W0, b0, g0, be0, W1, b1, g1, be1, W2, b2, g2, be2):
    loops = jnp.arange(N_NODES, dtype=edge_index.dtype)
    src = jnp.concatenate([edge_index[0], loops])
    dst = jnp.concatenate([edge_index[1], loops])
    key = lax.sort(dst * 16384 + src)
    key_p = jnp.concatenate(
        [key, jnp.full((KEY_PAD - E_TOT,), N_NODES * 16384, jnp.int32)])
    dst_s = key >> 14
    src_s = key & 16383
    src_p = jnp.concatenate(
        [src_s, jnp.zeros((E_PAD - E_TOT,), src_s.dtype)])
    wl_idx = jnp.minimum(
        (jnp.arange(E_PAD // EW, dtype=jnp.int32) + 1) * EW, E_TOT) - 1
    wlast = dst_s[wl_idx].astype(jnp.int32)
    wlast = jnp.concatenate([wlast, jnp.zeros((16,), jnp.int32)])
    rp2 = _rowptr_sc(key_p)
    rowptr = rp2[:RP_OUT] + rp2[RP_OUT:]
    rowptr = jnp.concatenate(
        [rowptr, jnp.zeros((PTR_PAD - RP_OUT,), jnp.int32)])

    h = x
    cnt2d = None
    for (W, b, g, be) in ((W0, b0, g0, be0), (W1, b1, g1, be1),
                          (W2, b2, g2, be2)):
        amax, amin, s1, s2, cnt = _sc_aggregate(h, src_p, wlast, rowptr)
        amax = amax.reshape(N_NODES, 128)
        amin = amin.reshape(N_NODES, 128)
        s1 = s1.reshape(N_NODES, 128)
        s2 = s2.reshape(N_NODES, 128)
        if cnt2d is None:
            cnt2d = jnp.broadcast_to(cnt.reshape(N_NODES, 16)[:, :1],
                                     (N_NODES, 128))
        h = _dense_layer(amax, amin, s1, s2, cnt2d, W, b, g, be)
    return h
